# Initial kernel scaffold; baseline (speedup 1.0000x reference)
import jax
import jax.numpy as jnp
from jax.experimental import pallas as pl


def kernel(x, edge_index, W_enc, b_enc, att1_W, att1_b, att2_W, att2_b, W_cls, b_cls):
    raise NotImplementedError("write your pallas kernel here")


# speedup vs baseline: 3.8463x; 3.8463x over previous
import functools
import jax
import jax.numpy as jnp
from jax import lax
from jax.experimental import pallas as pl
from jax.experimental.pallas import tpu as pltpu
from jax.experimental.pallas import tpu_sc as plsc

N = 10000
E = 320000
D = 128
EXT = 160
NC = 2
NS = 16
NW = NC * NS
EPW = E // NW
CH = 80
NCH = EPW // CH
NP = 10240
RPT = NP // NS
RB = 2000


def _sc_care(h_ext, d_rep, src, dst, zeros_ext):
    mesh = plsc.VectorSubcoreMesh(core_axis_name="c", subcore_axis_name="s")

    @functools.partial(
        pl.kernel,
        mesh=mesh,
        out_type=jax.ShapeDtypeStruct((NC, NP, EXT), jnp.float32),
        compiler_params=pltpu.CompilerParams(use_tc_tiling_on_sc=False),
        scratch_types=[
            pltpu.VMEM((CH,), jnp.int32),
            pltpu.VMEM((CH,), jnp.int32),
            pltpu.VMEM((CH, EXT), jnp.float32),
            pltpu.VMEM((CH, 16), jnp.float32),
            pltpu.VMEM_SHARED((NP, EXT), jnp.float32),
            pltpu.SemaphoreType.DMA,
            pltpu.SemaphoreType.DMA,
        ],
    )
    def k(h_hbm, d_hbm, src_hbm, dst_hbm, z_hbm, out_hbm,
          src_v, dst_v, rows_v, drows_v, accum, sem1, sem2):
        cid = lax.axis_index("c")
        sid = lax.axis_index("s")
        wid = sid * NC + cid

        pltpu.sync_copy(z_hbm.at[pl.ds(sid * RPT, RPT)],
                        accum.at[pl.ds(sid * RPT, RPT)])
        plsc.subcore_barrier()

        def chunk_body(g, carry):
            base = wid * EPW + g * CH
            pltpu.sync_copy(src_hbm.at[pl.ds(base, CH)], src_v)
            pltpu.sync_copy(dst_hbm.at[pl.ds(base, CH)], dst_v)
            cp1 = pltpu.async_copy(h_hbm.at[src_v], rows_v, sem1)
            cp2 = pltpu.async_copy(d_hbm.at[dst_v], drows_v, sem2)
            cp1.wait()
            cp2.wait()

            def scale_body(r, c2):
                z = drows_v[r] + rows_v[r, pl.ds(D + 16, 16)]
                av = 1.0 / (1.0 + jnp.exp(-z))
                for c in range(D // 16):
                    rows_v[r, pl.ds(c * 16, 16)] = (
                        rows_v[r, pl.ds(c * 16, 16)] * av)
                return c2

            lax.fori_loop(0, CH, scale_body, 0)
            pltpu.sync_copy(rows_v, accum.at[dst_v], add=True)
            return carry

        lax.fori_loop(0, NCH, chunk_body, 0)
        plsc.subcore_barrier()

        pltpu.sync_copy(accum.at[pl.ds(sid * RPT, RPT)],
                        out_hbm.at[cid, pl.ds(sid * RPT, RPT)])

    return k(h_ext, d_rep, src, dst, zeros_ext)


def _enc_body(x_ref, w_ref, b_ref, ap_ref, ab_ref, h_ref, ds_ref):
    h = jnp.maximum(
        jnp.dot(x_ref[...], w_ref[...], preferred_element_type=jnp.float32)
        + b_ref[...], 0.0)
    h_ref[...] = h
    ds_ref[...] = (
        jnp.dot(h, ap_ref[...], preferred_element_type=jnp.float32)
        + ab_ref[...])


def _mid_body(a0_ref, a1_ref, ap_ref, ab_ref, h_ref, ds_ref):
    a = a0_ref[...] + a1_ref[...]
    cnt = jnp.maximum(a[:, D:D + 1], 1.0)
    h = jnp.maximum(a[:, :D] / cnt, 0.0)
    h_ref[...] = h
    ds_ref[...] = (
        jnp.dot(h, ap_ref[...], preferred_element_type=jnp.float32)
        + ab_ref[...])


def _cls_body(a0_ref, a1_ref, w_ref, b_ref, out_ref):
    a = a0_ref[...] + a1_ref[...]
    cnt = jnp.maximum(a[:, D:D + 1], 1.0)
    m = a[:, :D] / cnt
    out_ref[...] = (
        jnp.dot(m, w_ref[...], preferred_element_type=jnp.float32)
        + b_ref[...])


def _att_pack(att_W, att_b):
    ap = jnp.zeros((D, D), jnp.float32)
    ap = ap.at[:, 0].set(att_W[:D, 0]).at[:, 1].set(att_W[D:, 0])
    ab = jnp.zeros((1, D), jnp.float32).at[0, 0].set(att_b[0])
    return ap, ab


def kernel(x, edge_index, W_enc, b_enc, att1_W, att1_b, att2_W, att2_b,
           W_cls, b_cls):
    src = edge_index[0]
    dst = edge_index[1]
    grid = (N // RB,)

    ap1, ab1 = _att_pack(att1_W, att1_b)
    ap2, ab2 = _att_pack(att2_W, att2_b)

    row_spec = pl.BlockSpec((RB, D), lambda i: (i, 0))
    ext_spec = pl.BlockSpec((RB, EXT), lambda i: (i, 0))
    w_spec = pl.BlockSpec((D, D), lambda i: (0, 0))
    b_spec = pl.BlockSpec((1, D), lambda i: (0, 0))

    h, ds1 = pl.pallas_call(
        _enc_body,
        grid=grid,
        in_specs=[row_spec, w_spec, b_spec, w_spec, b_spec],
        out_specs=[row_spec, row_spec],
        out_shape=[jax.ShapeDtypeStruct((N, D), jnp.float32),
                   jax.ShapeDtypeStruct((N, D), jnp.float32)],
    )(x, W_enc, b_enc.reshape(1, D), ap1, ab1)

    ones_pad = jnp.concatenate(
        [jnp.ones((N, 1), jnp.float32), jnp.zeros((N, 15), jnp.float32)],
        axis=1)
    zeros_ext = jnp.zeros((NP, EXT), jnp.float32)

    s1_rep = jnp.broadcast_to(ds1[:, 1:2], (N, 16))
    d1_rep = jnp.broadcast_to(ds1[:, 0:1], (N, 16))
    h_ext = jnp.concatenate([h, ones_pad, s1_rep], axis=1)
    agg1 = _sc_care(h_ext, d1_rep, src, dst, zeros_ext)[:, :N]

    h2, ds2 = pl.pallas_call(
        _mid_body,
        grid=grid,
        in_specs=[ext_spec, ext_spec, w_spec, b_spec],
        out_specs=[row_spec, row_spec],
        out_shape=[jax.ShapeDtypeStruct((N, D), jnp.float32),
                   jax.ShapeDtypeStruct((N, D), jnp.float32)],
    )(agg1[0], agg1[1], ap2, ab2)

    s2_rep = jnp.broadcast_to(ds2[:, 1:2], (N, 16))
    d2_rep = jnp.broadcast_to(ds2[:, 0:1], (N, 16))
    h2_ext = jnp.concatenate([h2, ones_pad, s2_rep], axis=1)
    agg2 = _sc_care(h2_ext, d2_rep, src, dst, zeros_ext)[:, :N]

    wc = jnp.zeros((D, D), jnp.float32).at[:, :2].set(W_cls)
    bc = jnp.zeros((1, D), jnp.float32).at[0, :2].set(b_cls)
    y = pl.pallas_call(
        _cls_body,
        grid=grid,
        in_specs=[ext_spec, ext_spec, w_spec, b_spec],
        out_specs=row_spec,
        out_shape=jax.ShapeDtypeStruct((N, D), jnp.float32),
    )(agg2[0], agg2[1], wc, bc)

    return y[:, :2]


# speedup vs baseline: 4.8341x; 1.2568x over previous
import functools
import jax
import jax.numpy as jnp
from jax import lax
from jax.experimental import pallas as pl
from jax.experimental.pallas import tpu as pltpu
from jax.experimental.pallas import tpu_sc as plsc

N = 10000
E = 320000
D = 128
EXT = 144
NC = 2
NS = 16
NW = NC * NS
EPW = E // NW
CH = 80
NCH = EPW // CH
NP = 10240
RPT = NP // NS
RB = 2000


def _sc_care(h_ext, d_rep, s_rep, src3, dst3, zeros_ext):
    mesh = plsc.VectorSubcoreMesh(core_axis_name="c", subcore_axis_name="s")

    @functools.partial(
        pl.kernel,
        mesh=mesh,
        out_type=jax.ShapeDtypeStruct((NC, NP, EXT), jnp.float32),
        compiler_params=pltpu.CompilerParams(use_tc_tiling_on_sc=False),
        scratch_types=[
            pltpu.VMEM((2, CH), jnp.int32),
            pltpu.VMEM((2, CH), jnp.int32),
            pltpu.VMEM((2, CH, EXT), jnp.float32),
            pltpu.VMEM((2, CH, 16), jnp.float32),
            pltpu.VMEM((2, CH, 16), jnp.float32),
            pltpu.VMEM_SHARED((NP, EXT), jnp.float32),
            pltpu.SemaphoreType.DMA,
            pltpu.SemaphoreType.DMA,
            pltpu.SemaphoreType.DMA,
            pltpu.SemaphoreType.DMA,
            pltpu.SemaphoreType.DMA,
            pltpu.SemaphoreType.DMA,
        ],
    )
    def k(h_hbm, d_hbm, s_hbm, src_hbm, dst_hbm, z_hbm, out_hbm,
          src_v, dst_v, rows_v, drows_v, srows_v, accum,
          gsem0, dsem0, ssem0, gsem1, dsem1, ssem1):
        cid = lax.axis_index("c")
        sid = lax.axis_index("s")
        wid = sid * NC + cid
        gsems = (gsem0, gsem1)
        dsems = (dsem0, dsem1)
        ssems = (ssem0, ssem1)

        pltpu.sync_copy(z_hbm.at[pl.ds(sid * RPT, RPT)],
                        accum.at[pl.ds(sid * RPT, RPT)])
        plsc.subcore_barrier()

        def issue(g, b):
            pltpu.sync_copy(src_hbm.at[wid, g], src_v.at[b])
            pltpu.sync_copy(dst_hbm.at[wid, g], dst_v.at[b])
            pltpu.async_copy(h_hbm.at[src_v.at[b]], rows_v.at[b], gsems[b])
            pltpu.async_copy(d_hbm.at[dst_v.at[b]], drows_v.at[b], dsems[b])
            pltpu.async_copy(s_hbm.at[src_v.at[b]], srows_v.at[b], ssems[b])

        def process(b):
            pltpu.make_async_copy(h_hbm.at[src_v.at[b]], rows_v.at[b],
                                  gsems[b]).wait()
            pltpu.make_async_copy(d_hbm.at[dst_v.at[b]], drows_v.at[b],
                                  dsems[b]).wait()
            pltpu.make_async_copy(s_hbm.at[src_v.at[b]], srows_v.at[b],
                                  ssems[b]).wait()

            def scale_body(r, c2):
                z = drows_v[b, r] + srows_v[b, r]
                av = 1.0 / (1.0 + jnp.exp(-z))
                for c in range(D // 16):
                    rows_v[b, r, pl.ds(c * 16, 16)] = (
                        rows_v[b, r, pl.ds(c * 16, 16)] * av)
                return c2

            lax.fori_loop(0, CH, scale_body, 0)
            pltpu.sync_copy(rows_v.at[b], accum.at[dst_v.at[b]], add=True)

        issue(0, 0)

        def pair_body(i, carry):
            issue(2 * i + 1, 1)
            process(0)
            issue(2 * i + 2, 0)
            process(1)
            return carry

        lax.fori_loop(0, (NCH - 1) // 2, pair_body, 0)
        process(0)
        plsc.subcore_barrier()

        pltpu.sync_copy(accum.at[pl.ds(sid * RPT, RPT)],
                        out_hbm.at[cid, pl.ds(sid * RPT, RPT)])

    return k(h_ext, d_rep, s_rep, src3, dst3, zeros_ext)


def _enc_body(x_ref, w_ref, b_ref, ap_ref, ab_ref, h_ref, ds_ref):
    h = jnp.maximum(
        jnp.dot(x_ref[...], w_ref[...], preferred_element_type=jnp.float32)
        + b_ref[...], 0.0)
    h_ref[...] = h
    ds_ref[...] = (
        jnp.dot(h, ap_ref[...], preferred_element_type=jnp.float32)
        + ab_ref[...])


def _mid_body(a0_ref, a1_ref, ap_ref, ab_ref, h_ref, ds_ref):
    a = a0_ref[...] + a1_ref[...]
    cnt = jnp.maximum(a[:, D:D + 1], 1.0)
    h = jnp.maximum(a[:, :D] / cnt, 0.0)
    h_ref[...] = h
    ds_ref[...] = (
        jnp.dot(h, ap_ref[...], preferred_element_type=jnp.float32)
        + ab_ref[...])


def _cls_body(a0_ref, a1_ref, w_ref, b_ref, out_ref):
    a = a0_ref[...] + a1_ref[...]
    cnt = jnp.maximum(a[:, D:D + 1], 1.0)
    m = a[:, :D] / cnt
    out_ref[...] = (
        jnp.dot(m, w_ref[...], preferred_element_type=jnp.float32)
        + b_ref[...])


def _att_pack(att_W, att_b):
    ap = jnp.zeros((D, D), jnp.float32)
    ap = ap.at[:, 0].set(att_W[:D, 0]).at[:, 1].set(att_W[D:, 0])
    ab = jnp.zeros((1, D), jnp.float32).at[0, 0].set(att_b[0])
    return ap, ab


def kernel(x, edge_index, W_enc, b_enc, att1_W, att1_b, att2_W, att2_b,
           W_cls, b_cls):
    src = edge_index[0].reshape(NW, NCH, CH)
    dst = edge_index[1].reshape(NW, NCH, CH)
    grid = (N // RB,)

    ap1, ab1 = _att_pack(att1_W, att1_b)
    ap2, ab2 = _att_pack(att2_W, att2_b)

    row_spec = pl.BlockSpec((RB, D), lambda i: (i, 0))
    ext_spec = pl.BlockSpec((RB, EXT), lambda i: (i, 0))
    w_spec = pl.BlockSpec((D, D), lambda i: (0, 0))
    b_spec = pl.BlockSpec((1, D), lambda i: (0, 0))

    h, ds1 = pl.pallas_call(
        _enc_body,
        grid=grid,
        in_specs=[row_spec, w_spec, b_spec, w_spec, b_spec],
        out_specs=[row_spec, row_spec],
        out_shape=[jax.ShapeDtypeStruct((N, D), jnp.float32),
                   jax.ShapeDtypeStruct((N, D), jnp.float32)],
    )(x, W_enc, b_enc.reshape(1, D), ap1, ab1)

    ones_pad = jnp.concatenate(
        [jnp.ones((N, 1), jnp.float32), jnp.zeros((N, 15), jnp.float32)],
        axis=1)
    zeros_ext = jnp.zeros((NP, EXT), jnp.float32)

    s1_rep = jnp.broadcast_to(ds1[:, 1:2], (N, 16))
    d1_rep = jnp.broadcast_to(ds1[:, 0:1], (N, 16))
    h_ext = jnp.concatenate([h, ones_pad], axis=1)
    agg1 = _sc_care(h_ext, d1_rep, s1_rep, src, dst, zeros_ext)[:, :N]

    h2, ds2 = pl.pallas_call(
        _mid_body,
        grid=grid,
        in_specs=[ext_spec, ext_spec, w_spec, b_spec],
        out_specs=[row_spec, row_spec],
        out_shape=[jax.ShapeDtypeStruct((N, D), jnp.float32),
                   jax.ShapeDtypeStruct((N, D), jnp.float32)],
    )(agg1[0], agg1[1], ap2, ab2)

    s2_rep = jnp.broadcast_to(ds2[:, 1:2], (N, 16))
    d2_rep = jnp.broadcast_to(ds2[:, 0:1], (N, 16))
    h2_ext = jnp.concatenate([h2, ones_pad], axis=1)
    agg2 = _sc_care(h2_ext, d2_rep, s2_rep, src, dst, zeros_ext)[:, :N]

    wc = jnp.zeros((D, D), jnp.float32).at[:, :2].set(W_cls)
    bc = jnp.zeros((1, D), jnp.float32).at[0, :2].set(b_cls)
    y = pl.pallas_call(
        _cls_body,
        grid=grid,
        in_specs=[ext_spec, ext_spec, w_spec, b_spec],
        out_specs=row_spec,
        out_shape=jax.ShapeDtypeStruct((N, D), jnp.float32),
    )(agg2[0], agg2[1], wc, bc)

    return y[:, :2]


# speedup vs baseline: 5.9642x; 1.2338x over previous
import functools
import jax
import jax.numpy as jnp
from jax import lax
from jax.experimental import pallas as pl
from jax.experimental.pallas import tpu as pltpu
from jax.experimental.pallas import tpu_sc as plsc

N = 10000
E = 320000
D = 128
EXT = 144
NC = 2
NS = 16
NW = NC * NS
EPW = E // NW
CH = 80
NCH = EPW // CH
NP = 10112
RPT = NP // NS
RB = 2000


def _sc_care(h_ext, d_rep, s_rep, pk3, zeros_ext):
    mesh = plsc.VectorSubcoreMesh(core_axis_name="c", subcore_axis_name="s")

    @functools.partial(
        pl.kernel,
        mesh=mesh,
        out_type=jax.ShapeDtypeStruct((NC, NP, EXT), jnp.float32),
        compiler_params=pltpu.CompilerParams(use_tc_tiling_on_sc=False),
        scratch_types=[
            pltpu.VMEM((EPW,), jnp.int32),
            pltpu.VMEM((2, CH), jnp.int32),
            pltpu.VMEM((2, CH), jnp.int32),
            pltpu.VMEM((2, CH, EXT), jnp.float32),
            pltpu.VMEM((2, CH, 16), jnp.float32),
            pltpu.VMEM((2, CH, 16), jnp.float32),
            pltpu.VMEM_SHARED((NP, EXT), jnp.float32),
            pltpu.SemaphoreType.DMA,
            pltpu.SemaphoreType.DMA,
            pltpu.SemaphoreType.DMA,
            pltpu.SemaphoreType.DMA,
            pltpu.SemaphoreType.DMA,
            pltpu.SemaphoreType.DMA,
        ],
    )
    def k(h_hbm, d_hbm, s_hbm, pk_hbm, z_hbm, out_hbm,
          pk_v, src_v, dst_v, rows_v, drows_v, srows_v, accum,
          gsem0, dsem0, ssem0, gsem1, dsem1, ssem1):
        cid = lax.axis_index("c")
        sid = lax.axis_index("s")
        wid = sid * NC + cid
        gsems = (gsem0, gsem1)
        dsems = (dsem0, dsem1)
        ssems = (ssem0, ssem1)

        pltpu.sync_copy(pk_hbm.at[wid], pk_v)
        pltpu.sync_copy(z_hbm.at[pl.ds(sid * RPT, RPT)],
                        accum.at[pl.ds(sid * RPT, RPT)])
        plsc.subcore_barrier()

        def issue(g, b):
            def decode(i, c2):
                v = pk_v[pl.ds(g * CH + i * 16, 16)]
                src_v[b, pl.ds(i * 16, 16)] = jnp.bitwise_and(v, 16383)
                dst_v[b, pl.ds(i * 16, 16)] = lax.shift_right_logical(v, 14)
                return c2

            lax.fori_loop(0, CH // 16, decode, 0)
            pltpu.async_copy(h_hbm.at[src_v.at[b]], rows_v.at[b], gsems[b])
            pltpu.async_copy(d_hbm.at[dst_v.at[b]], drows_v.at[b], dsems[b])
            pltpu.async_copy(s_hbm.at[src_v.at[b]], srows_v.at[b], ssems[b])

        def process(b):
            pltpu.make_async_copy(h_hbm.at[src_v.at[b]], rows_v.at[b],
                                  gsems[b]).wait()
            pltpu.make_async_copy(d_hbm.at[dst_v.at[b]], drows_v.at[b],
                                  dsems[b]).wait()
            pltpu.make_async_copy(s_hbm.at[src_v.at[b]], srows_v.at[b],
                                  ssems[b]).wait()

            def scale_body(r, c2):
                z = drows_v[b, r] + srows_v[b, r]
                av = 1.0 / (1.0 + jnp.exp(-z))
                for c in range(D // 16):
                    rows_v[b, r, pl.ds(c * 16, 16)] = (
                        rows_v[b, r, pl.ds(c * 16, 16)] * av)
                return c2

            lax.fori_loop(0, CH, scale_body, 0)
            pltpu.sync_copy(rows_v.at[b], accum.at[dst_v.at[b]], add=True)

        issue(0, 0)

        def pair_body(i, carry):
            issue(2 * i + 1, 1)
            process(0)
            issue(2 * i + 2, 0)
            process(1)
            return carry

        lax.fori_loop(0, (NCH - 1) // 2, pair_body, 0)
        process(0)
        plsc.subcore_barrier()

        pltpu.sync_copy(accum.at[pl.ds(sid * RPT, RPT)],
                        out_hbm.at[cid, pl.ds(sid * RPT, RPT)])

    return k(h_ext, d_rep, s_rep, pk3, zeros_ext)


def _enc_body(x_ref, w_ref, b_ref, ap_ref, ab_ref, h_ref, ds_ref):
    h = jnp.maximum(
        jnp.dot(x_ref[...], w_ref[...], preferred_element_type=jnp.float32)
        + b_ref[...], 0.0)
    h_ref[...] = h
    ds_ref[...] = (
        jnp.dot(h, ap_ref[...], preferred_element_type=jnp.float32)
        + ab_ref[...])


def _mid_body(a0_ref, a1_ref, ap_ref, ab_ref, h_ref, ds_ref):
    a = a0_ref[...] + a1_ref[...]
    cnt = jnp.maximum(a[:, D:D + 1], 1.0)
    h = jnp.maximum(a[:, :D] / cnt, 0.0)
    h_ref[...] = h
    ds_ref[...] = (
        jnp.dot(h, ap_ref[...], preferred_element_type=jnp.float32)
        + ab_ref[...])


def _cls_body(a0_ref, a1_ref, w_ref, b_ref, out_ref):
    a = a0_ref[...] + a1_ref[...]
    cnt = jnp.maximum(a[:, D:D + 1], 1.0)
    m = a[:, :D] / cnt
    out_ref[...] = (
        jnp.dot(m, w_ref[...], preferred_element_type=jnp.float32)
        + b_ref[...])


def _att_pack(att_W, att_b):
    ap = jnp.zeros((D, D), jnp.float32)
    ap = ap.at[:, 0].set(att_W[:D, 0]).at[:, 1].set(att_W[D:, 0])
    ab = jnp.zeros((1, D), jnp.float32).at[0, 0].set(att_b[0])
    return ap, ab


def kernel(x, edge_index, W_enc, b_enc, att1_W, att1_b, att2_W, att2_b,
           W_cls, b_cls):
    packed = (edge_index[0] + edge_index[1] * 16384).reshape(NW, EPW)
    grid = (N // RB,)

    ap1, ab1 = _att_pack(att1_W, att1_b)
    ap2, ab2 = _att_pack(att2_W, att2_b)

    row_spec = pl.BlockSpec((RB, D), lambda i: (i, 0))
    ext_spec = pl.BlockSpec((RB, EXT), lambda i: (i, 0))
    w_spec = pl.BlockSpec((D, D), lambda i: (0, 0))
    b_spec = pl.BlockSpec((1, D), lambda i: (0, 0))

    h, ds1 = pl.pallas_call(
        _enc_body,
        grid=grid,
        in_specs=[row_spec, w_spec, b_spec, w_spec, b_spec],
        out_specs=[row_spec, row_spec],
        out_shape=[jax.ShapeDtypeStruct((N, D), jnp.float32),
                   jax.ShapeDtypeStruct((N, D), jnp.float32)],
    )(x, W_enc, b_enc.reshape(1, D), ap1, ab1)

    ones_pad = jnp.concatenate(
        [jnp.ones((N, 1), jnp.float32), jnp.zeros((N, 15), jnp.float32)],
        axis=1)
    zeros_ext = jnp.zeros((NP, EXT), jnp.float32)

    s1_rep = jnp.broadcast_to(ds1[:, 1:2], (N, 16))
    d1_rep = jnp.broadcast_to(ds1[:, 0:1], (N, 16))
    h_ext = jnp.concatenate([h, ones_pad], axis=1)
    agg1 = _sc_care(h_ext, d1_rep, s1_rep, packed, zeros_ext)[:, :N]

    h2, ds2 = pl.pallas_call(
        _mid_body,
        grid=grid,
        in_specs=[ext_spec, ext_spec, w_spec, b_spec],
        out_specs=[row_spec, row_spec],
        out_shape=[jax.ShapeDtypeStruct((N, D), jnp.float32),
                   jax.ShapeDtypeStruct((N, D), jnp.float32)],
    )(agg1[0], agg1[1], ap2, ab2)

    s2_rep = jnp.broadcast_to(ds2[:, 1:2], (N, 16))
    d2_rep = jnp.broadcast_to(ds2[:, 0:1], (N, 16))
    h2_ext = jnp.concatenate([h2, ones_pad], axis=1)
    agg2 = _sc_care(h2_ext, d2_rep, s2_rep, packed, zeros_ext)[:, :N]

    wc = jnp.zeros((D, D), jnp.float32).at[:, :2].set(W_cls)
    bc = jnp.zeros((1, D), jnp.float32).at[0, :2].set(b_cls)
    y = pl.pallas_call(
        _cls_body,
        grid=grid,
        in_specs=[ext_spec, ext_spec, w_spec, b_spec],
        out_specs=row_spec,
        out_shape=jax.ShapeDtypeStruct((N, D), jnp.float32),
    )(agg2[0], agg2[1], wc, bc)

    return y[:, :2]
